# baseline (speedup 1.0000x reference)
import functools

import jax
import jax.numpy as jnp
from jax import lax
from jax.experimental import pallas as pl
from jax.experimental.pallas import tpu as pltpu
from jax.experimental.pallas import tpu_sc as plsc

NC, NS, LANES = 2, 16, 16
NW = NC * NS
NQ = 4
QW = 64

_MESH = plsc.VectorSubcoreMesh(core_axis_name="c", subcore_axis_name="s")

f32 = jnp.float32


def _node_mm_body(h_ref, wa, ba, wb, bb, wd, bd, wem, bem, ah_ref, *q_refs):
    hb = h_ref[...]
    ah_ref[...] = jnp.dot(hb, wa[...], preferred_element_type=f32) + ba[...]
    bh = jnp.dot(hb, wb[...], preferred_element_type=f32) + bb[...]
    dh = jnp.dot(hb, wd[...], preferred_element_type=f32) + bd[...]
    eh = jnp.dot(hb, wem[...], preferred_element_type=f32) + bem[...]
    for q in range(NQ):
        q_refs[q][...] = bh[:, q * QW:(q + 1) * QW]
        q_refs[NQ + q][...] = dh[:, q * QW:(q + 1) * QW]
        q_refs[2 * NQ + q][...] = eh[:, q * QW:(q + 1) * QW]


def _node_mms(h, WA, bA, WB, bB, WD, bD, WEm, bEm):
    N, D = h.shape
    BN = 2000
    row_spec = pl.BlockSpec((BN, D), lambda i: (i, 0))
    w_spec = pl.BlockSpec((D, D), lambda i: (0, 0))
    b_spec = pl.BlockSpec((1, D), lambda i: (0, 0))
    q_spec = pl.BlockSpec((BN, QW), lambda i: (i, 0))
    q_shape = jax.ShapeDtypeStruct((N, QW), f32)
    outs = pl.pallas_call(
        _node_mm_body,
        grid=(N // BN,),
        in_specs=[row_spec, w_spec, b_spec, w_spec, b_spec, w_spec, b_spec,
                  w_spec, b_spec],
        out_specs=[row_spec] + [q_spec] * (3 * NQ),
        out_shape=[jax.ShapeDtypeStruct((N, D), f32)] + [q_shape] * (3 * NQ),
    )(h, WA, bA.reshape(1, D), WB, bB.reshape(1, D), WD, bD.reshape(1, D),
      WEm, bEm.reshape(1, D))
    ah = outs[0]
    bh_q = outs[1:1 + NQ]
    dh_q = outs[1 + NQ:1 + 2 * NQ]
    eh_q = outs[1 + 2 * NQ:1 + 3 * NQ]
    return ah, bh_q, dh_q, eh_q


def _edge_mm_body(e_ref, wc, bc, *q_refs):
    ce = jnp.dot(e_ref[...], wc[...], preferred_element_type=f32) + bc[...]
    for q in range(NQ):
        q_refs[q][...] = ce[:, q * QW:(q + 1) * QW]


def _edge_mm(e, WC, bC):
    E, D = e.shape
    BE = 2000
    return pl.pallas_call(
        _edge_mm_body,
        grid=(E // BE,),
        in_specs=[pl.BlockSpec((BE, D), lambda i: (i, 0)),
                  pl.BlockSpec((D, D), lambda i: (0, 0)),
                  pl.BlockSpec((1, D), lambda i: (0, 0))],
        out_specs=[pl.BlockSpec((BE, QW), lambda i: (i, 0))] * NQ,
        out_shape=[jax.ShapeDtypeStruct((E, QW), f32)] * NQ,
    )(e, WC, bC.reshape(1, D))


def _sc_fused_body(src_hbm, dst_hbm, *refs, NP, E, CH):
    dh_q = refs[0:4]
    eh_q = refs[4:8]
    bh_q = refs[8:12]
    ce_q = refs[12:16]
    ep_q = refs[16:20]
    sg_q = refs[20:24]
    sh_q = refs[24:28]
    sum_st = refs[28]
    sumsq_st = refs[29]
    (sidx, didx, dbuf, ebuf, cbuf, bbuf, stbuf, stidx, zbuf,
     s0, s1, s2, acc_sig, acc_sh, acc_st) = refs[30:]

    cid = lax.axis_index("c")
    sid = lax.axis_index("s")
    nchunks = E // CH
    per = (nchunks + NS - 1) // NS
    rows_pt = NP // NS
    nz = rows_pt // 8

    def zrow(r, cc):
        for k in range(QW // 16):
            zbuf[r, pl.ds(k * 16, 16)] = jnp.zeros((16,), f32)
        return cc

    lax.fori_loop(0, 8, zrow, 0)
    stidx[...] = lax.iota(jnp.int32, 16)

    def zero_stbuf():
        def zr(r, cc):
            for k in range(QW // 16):
                stbuf[r, pl.ds(k * 16, 16)] = jnp.zeros((16,), f32)
            return cc
        lax.fori_loop(0, 16, zr, 0)

    def zero_accs():
        def zc(qq, cc):
            pltpu.sync_copy(zbuf, acc_sig.at[pl.ds(sid * rows_pt + qq * 8, 8)])
            pltpu.sync_copy(zbuf, acc_sh.at[pl.ds(sid * rows_pt + qq * 8, 8)])
            return cc
        lax.fori_loop(0, nz, zc, 0)

    def round_q(q):
        dh, eh, bh, ce, ep, sg, sh = (dh_q[q], eh_q[q], bh_q[q], ce_q[q],
                                      ep_q[q], sg_q[q], sh_q[q])
        zero_accs()
        zero_stbuf()

        @pl.when(sid == 0)
        def _():
            pltpu.sync_copy(stbuf, acc_st)

        plsc.subcore_barrier()

        def chunk_body(j, carry):
            chunk = sid + j * NS

            @pl.when(chunk < nchunks)
            def _():
                base = chunk * CH
                pltpu.sync_copy(src_hbm.at[pl.ds(base, CH)], sidx)
                pltpu.sync_copy(dst_hbm.at[pl.ds(base, CH)], didx)
                c0 = pltpu.async_copy(dh.at[sidx], dbuf, s0)
                c1 = pltpu.async_copy(eh.at[didx], ebuf, s1)
                c2 = pltpu.async_copy(ce.at[pl.ds(base, CH)], cbuf, s2)
                c0.wait()
                c1.wait()
                c2.wait()
                c3 = pltpu.async_copy(bh.at[sidx], bbuf, s0)

                def row(r, st):
                    nst = []
                    for k in range(QW // 16):
                        sl = pl.ds(k * 16, 16)
                        epv = dbuf[r, sl] + ebuf[r, sl] + cbuf[r, sl]
                        dbuf[r, sl] = epv
                        nst.append(st[k] + epv)
                        nst.append(st[4 + k] + epv * epv)
                        sig = 1.0 / (1.0 + jnp.exp(-epv))
                        ebuf[r, sl] = sig
                    return tuple(nst[::2]) + tuple(nst[1::2])

                z16 = jnp.zeros((16,), f32)
                st = lax.fori_loop(0, CH, row, (z16,) * 8)
                c3.wait()

                def row2(r, cc):
                    for k in range(QW // 16):
                        sl = pl.ds(k * 16, 16)
                        cbuf[r, sl] = ebuf[r, sl] * bbuf[r, sl]
                    return cc

                lax.fori_loop(0, CH, row2, 0)
                for k in range(QW // 16):
                    sl = pl.ds(k * 16, 16)
                    stbuf[0, sl] = stbuf[0, sl] + st[k]
                    stbuf[1, sl] = stbuf[1, sl] + st[4 + k]
                pltpu.sync_copy(dbuf, ep.at[pl.ds(base, CH)])
                pltpu.sync_copy(ebuf, acc_sig.at[didx], add=True)
                pltpu.sync_copy(cbuf, acc_sh.at[didx], add=True)

            return carry

        lax.fori_loop(0, per, chunk_body, 0)
        pltpu.sync_copy(stbuf, acc_st.at[stidx], add=True)
        plsc.subcore_barrier()
        pltpu.sync_copy(acc_sig.at[pl.ds(sid * rows_pt, rows_pt)],
                        sg.at[pl.ds(sid * rows_pt, rows_pt)])
        pltpu.sync_copy(acc_sh.at[pl.ds(sid * rows_pt, rows_pt)],
                        sh.at[pl.ds(sid * rows_pt, rows_pt)])

        @pl.when(sid == 0)
        def _():
            pltpu.sync_copy(acc_st.at[pl.ds(0, 1)], sum_st.at[pl.ds(q, 1)])
            pltpu.sync_copy(acc_st.at[pl.ds(1, 1)], sumsq_st.at[pl.ds(q, 1)])

    for r in range(2):
        @pl.when(cid == 0)
        def _(r=r):
            round_q(2 * r)

        @pl.when(cid == 1)
        def _(r=r):
            round_q(2 * r + 1)


def _sc_fused(src, dst, dh_q, eh_q, bh_q, ce_q):
    E = src.shape[0]
    N = dh_q[0].shape[0]
    unit = NS * 8
    NP = ((N + unit - 1) // unit) * unit
    CH = 64
    q_ep = jax.ShapeDtypeStruct((E, QW), f32)
    q_acc = jax.ShapeDtypeStruct((NP, QW), f32)
    sk = pl.kernel(
        functools.partial(_sc_fused_body, NP=NP, E=E, CH=CH),
        out_type=([q_ep] * NQ + [q_acc] * NQ + [q_acc] * NQ
                  + [jax.ShapeDtypeStruct((NQ, QW), f32)] * 2),
        mesh=_MESH,
        compiler_params=pltpu.CompilerParams(use_tc_tiling_on_sc=False),
        scratch_types=[
            pltpu.VMEM((CH,), jnp.int32),
            pltpu.VMEM((CH,), jnp.int32),
            pltpu.VMEM((CH, QW), f32),
            pltpu.VMEM((CH, QW), f32),
            pltpu.VMEM((CH, QW), f32),
            pltpu.VMEM((CH, QW), f32),
            pltpu.VMEM((16, QW), f32),
            pltpu.VMEM((16,), jnp.int32),
            pltpu.VMEM((8, QW), f32),
            pltpu.SemaphoreType.DMA,
            pltpu.SemaphoreType.DMA,
            pltpu.SemaphoreType.DMA,
            pltpu.VMEM_SHARED((NP, QW), f32),
            pltpu.VMEM_SHARED((NP, QW), f32),
            pltpu.VMEM_SHARED((16, QW), f32),
        ],
    )
    outs = sk(src, dst, *dh_q, *eh_q, *bh_q, *ce_q)
    ep_q = outs[0:NQ]
    sg_q = outs[NQ:2 * NQ]
    sh_q = outs[2 * NQ:3 * NQ]
    sum_st, sumsq_st = outs[3 * NQ], outs[3 * NQ + 1]
    return ep_q, sg_q, sh_q, sum_st, sumsq_st


def _fin_h_a_body(ah_ref, *refs):
    sg_refs = refs[0:NQ]
    sh_refs = refs[NQ:2 * NQ]
    hnew_ref, st_ref = refs[2 * NQ], refs[2 * NQ + 1]
    i = pl.program_id(0)
    ssig = jnp.concatenate([r[...] for r in sg_refs], axis=1)
    ssh = jnp.concatenate([r[...] for r in sh_refs], axis=1)
    hn = ah_ref[...] + ssh / (ssig + 1e-6)
    hnew_ref[...] = hn
    s1 = jnp.sum(hn, axis=0, keepdims=True)
    s2 = jnp.sum(hn * hn, axis=0, keepdims=True)
    blk = jnp.concatenate([s1, s2], axis=0)

    @pl.when(i == 0)
    def _():
        st_ref[...] = blk

    @pl.when(i > 0)
    def _():
        st_ref[...] = st_ref[...] + blk


def _fin_h_a(ah, sg_q, sh_q):
    N, D = ah.shape
    BN = 2000
    row = pl.BlockSpec((BN, D), lambda i: (i, 0))
    qspec = pl.BlockSpec((BN, QW), lambda i: (i, 0))
    return pl.pallas_call(
        _fin_h_a_body,
        grid=(N // BN,),
        in_specs=[row] + [qspec] * (2 * NQ),
        out_specs=[row, pl.BlockSpec((2, D), lambda i: (0, 0))],
        out_shape=[jax.ShapeDtypeStruct((N, D), f32),
                   jax.ShapeDtypeStruct((2, D), f32)],
    )(ah, *sg_q, *sh_q)


def _fin_h_b_body(x_ref, xn_ref, st_ref, g_ref, b_ref, out_ref, *, count):
    mean = st_ref[0:1, :] / count
    var = st_ref[1:2, :] / count - mean * mean
    inv = lax.rsqrt(var + 1e-5)
    xn = (xn_ref[...] - mean) * inv * g_ref[...] + b_ref[...]
    out_ref[...] = x_ref[...] + jnp.maximum(xn, 0.0)


def _fin_h_b(h, h_new, st, gamma, beta):
    N, D = h.shape
    BN = 2000
    row = pl.BlockSpec((BN, D), lambda i: (i, 0))
    return pl.pallas_call(
        functools.partial(_fin_h_b_body, count=float(N)),
        grid=(N // BN,),
        in_specs=[row, row, pl.BlockSpec((2, D), lambda i: (0, 0)),
                  pl.BlockSpec((1, D), lambda i: (0, 0)),
                  pl.BlockSpec((1, D), lambda i: (0, 0))],
        out_specs=row,
        out_shape=jax.ShapeDtypeStruct((N, D), f32),
    )(h, h_new, st, gamma.reshape(1, D), beta.reshape(1, D))


def _fin_e_b_body(e_ref, *refs, count):
    ep_refs = refs[0:NQ]
    s1_ref, s2_ref, g_ref, b_ref, out_ref = refs[NQ:]
    parts = []
    for q in range(NQ):
        mean = s1_ref[q:q + 1, :] / count
        var = s2_ref[q:q + 1, :] / count - mean * mean
        inv = lax.rsqrt(var + 1e-5)
        parts.append((ep_refs[q][...] - mean) * inv)
    xn = jnp.concatenate(parts, axis=1) * g_ref[...] + b_ref[...]
    out_ref[...] = e_ref[...] + jnp.maximum(xn, 0.0)


def _fin_e_b(e, ep_q, sum_st, sumsq_st, gamma, beta):
    E, D = e.shape
    BE = 2000
    row = pl.BlockSpec((BE, D), lambda i: (i, 0))
    qspec = pl.BlockSpec((BE, QW), lambda i: (i, 0))
    st_spec = pl.BlockSpec((NQ, QW), lambda i: (0, 0))
    return pl.pallas_call(
        functools.partial(_fin_e_b_body, count=float(E)),
        grid=(E // BE,),
        in_specs=[row] + [qspec] * NQ + [st_spec, st_spec,
                  pl.BlockSpec((1, D), lambda i: (0, 0)),
                  pl.BlockSpec((1, D), lambda i: (0, 0))],
        out_specs=row,
        out_shape=jax.ShapeDtypeStruct((E, D), f32),
    )(e, *ep_q, sum_st, sumsq_st, gamma.reshape(1, D), beta.reshape(1, D))


def kernel(h, e, edge_index, WA, bA, WB, bB, WC, bC, WD, bD, WEm, bEm,
           bn_gh, bn_bh, bn_ge, bn_be):
    src = edge_index[0]
    dst = edge_index[1]
    ah, bh_q, dh_q, eh_q = _node_mms(h, WA, bA, WB, bB, WD, bD, WEm, bEm)
    ce_q = _edge_mm(e, WC, bC)
    ep_q, sg_q, sh_q, sum_st, sumsq_st = _sc_fused(src, dst, dh_q, eh_q,
                                                   bh_q, ce_q)
    h_new, st_h = _fin_h_a(ah, sg_q, sh_q)
    h_out = _fin_h_b(h, h_new, st_h, bn_gh, bn_bh)
    e_out = _fin_e_b(e, ep_q, sum_st, sumsq_st, bn_ge, bn_be)
    return (h_out, e_out)


# speedup vs baseline: 1.1475x; 1.1475x over previous
import functools

import jax
import jax.numpy as jnp
from jax import lax
from jax.experimental import pallas as pl
from jax.experimental.pallas import tpu as pltpu
from jax.experimental.pallas import tpu_sc as plsc

NC, NS, LANES = 2, 16, 16
NW = NC * NS
NQ = 4
QW = 64

_MESH = plsc.VectorSubcoreMesh(core_axis_name="c", subcore_axis_name="s")

f32 = jnp.float32


def _node_mm_body(h_ref, wa, ba, wb, bb, wd, bd, wem, bem, ah_ref, *q_refs):
    hb = h_ref[...]
    ah_ref[...] = jnp.dot(hb, wa[...], preferred_element_type=f32) + ba[...]
    bh = jnp.dot(hb, wb[...], preferred_element_type=f32) + bb[...]
    dh = jnp.dot(hb, wd[...], preferred_element_type=f32) + bd[...]
    eh = jnp.dot(hb, wem[...], preferred_element_type=f32) + bem[...]
    for q in range(NQ):
        q_refs[q][...] = bh[:, q * QW:(q + 1) * QW]
        q_refs[NQ + q][...] = dh[:, q * QW:(q + 1) * QW]
        q_refs[2 * NQ + q][...] = eh[:, q * QW:(q + 1) * QW]


def _node_mms(h, WA, bA, WB, bB, WD, bD, WEm, bEm):
    N, D = h.shape
    BN = 2000
    row_spec = pl.BlockSpec((BN, D), lambda i: (i, 0))
    w_spec = pl.BlockSpec((D, D), lambda i: (0, 0))
    b_spec = pl.BlockSpec((1, D), lambda i: (0, 0))
    q_spec = pl.BlockSpec((BN, QW), lambda i: (i, 0))
    q_shape = jax.ShapeDtypeStruct((N, QW), f32)
    outs = pl.pallas_call(
        _node_mm_body,
        grid=(N // BN,),
        in_specs=[row_spec, w_spec, b_spec, w_spec, b_spec, w_spec, b_spec,
                  w_spec, b_spec],
        out_specs=[row_spec] + [q_spec] * (3 * NQ),
        out_shape=[jax.ShapeDtypeStruct((N, D), f32)] + [q_shape] * (3 * NQ),
    )(h, WA, bA.reshape(1, D), WB, bB.reshape(1, D), WD, bD.reshape(1, D),
      WEm, bEm.reshape(1, D))
    ah = outs[0]
    bh_q = outs[1:1 + NQ]
    dh_q = outs[1 + NQ:1 + 2 * NQ]
    eh_q = outs[1 + 2 * NQ:1 + 3 * NQ]
    return ah, bh_q, dh_q, eh_q


def _edge_mm_body(e_ref, wc, bc, ce_ref):
    ce_ref[...] = jnp.dot(e_ref[...], wc[...], preferred_element_type=f32) + bc[...]


def _edge_mm(e, WC, bC):
    E, D = e.shape
    BE = 2000
    return pl.pallas_call(
        _edge_mm_body,
        grid=(E // BE,),
        in_specs=[pl.BlockSpec((BE, D), lambda i: (i, 0)),
                  pl.BlockSpec((D, D), lambda i: (0, 0)),
                  pl.BlockSpec((1, D), lambda i: (0, 0))],
        out_specs=pl.BlockSpec((BE, D), lambda i: (i, 0)),
        out_shape=jax.ShapeDtypeStruct((E, D), f32),
    )(e, WC, bC.reshape(1, D))


def _sc_fused_body(src_hbm, dst_hbm, *refs, NP, E, CH):
    dh_q = refs[0:4]
    eh_q = refs[4:8]
    bh_q = refs[8:12]
    ce_hbm = refs[12]
    ep_hbm = refs[13]
    sg_q = refs[14:18]
    sh_q = refs[18:22]
    sum_st = refs[22]
    sumsq_st = refs[23]
    (sidx, didx, dbuf, ebuf, cbuf, bbuf, stbuf, stidx, zbuf,
     s0, s1, s2, acc_sig, acc_sh, acc_st) = refs[24:]

    cid = lax.axis_index("c")
    sid = lax.axis_index("s")
    nchunks = E // CH
    per = (nchunks + NS - 1) // NS
    rows_pt = NP // NS
    nz = rows_pt // 8

    def zrow(r, cc):
        for k in range(QW // 16):
            zbuf[r, pl.ds(k * 16, 16)] = jnp.zeros((16,), f32)
        return cc

    lax.fori_loop(0, 8, zrow, 0)
    stidx[...] = lax.iota(jnp.int32, 16)

    def zero_stbuf():
        def zr(r, cc):
            for k in range(QW // 16):
                stbuf[r, pl.ds(k * 16, 16)] = jnp.zeros((16,), f32)
            return cc
        lax.fori_loop(0, 16, zr, 0)

    def zero_accs():
        def zc(qq, cc):
            pltpu.sync_copy(zbuf, acc_sig.at[pl.ds(sid * rows_pt + qq * 8, 8)])
            pltpu.sync_copy(zbuf, acc_sh.at[pl.ds(sid * rows_pt + qq * 8, 8)])
            return cc
        lax.fori_loop(0, nz, zc, 0)

    def round_q(q):
        dh, eh, bh, sg, sh = dh_q[q], eh_q[q], bh_q[q], sg_q[q], sh_q[q]
        col = q * QW
        zero_accs()
        zero_stbuf()

        @pl.when(sid == 0)
        def _():
            pltpu.sync_copy(stbuf, acc_st)

        plsc.subcore_barrier()

        def chunk_body(j, carry):
            chunk = sid + j * NS

            @pl.when(chunk < nchunks)
            def _():
                base = chunk * CH
                pltpu.sync_copy(src_hbm.at[pl.ds(base, CH)], sidx)
                pltpu.sync_copy(dst_hbm.at[pl.ds(base, CH)], didx)
                c0 = pltpu.async_copy(dh.at[sidx], dbuf, s0)
                c1 = pltpu.async_copy(eh.at[didx], ebuf, s1)
                c2 = pltpu.async_copy(
                    ce_hbm.at[pl.ds(base, CH), pl.ds(col, QW)], cbuf, s2)
                c0.wait()
                c1.wait()
                c2.wait()
                c3 = pltpu.async_copy(bh.at[sidx], bbuf, s0)

                def row(r, st):
                    nst = []
                    for k in range(QW // 16):
                        sl = pl.ds(k * 16, 16)
                        epv = dbuf[r, sl] + ebuf[r, sl] + cbuf[r, sl]
                        dbuf[r, sl] = epv
                        nst.append(st[k] + epv)
                        nst.append(st[4 + k] + epv * epv)
                        sig = 1.0 / (1.0 + jnp.exp(-epv))
                        ebuf[r, sl] = sig
                    return tuple(nst[::2]) + tuple(nst[1::2])

                z16 = jnp.zeros((16,), f32)
                st = lax.fori_loop(0, CH, row, (z16,) * 8)
                c3.wait()

                def row2(r, cc):
                    for k in range(QW // 16):
                        sl = pl.ds(k * 16, 16)
                        cbuf[r, sl] = ebuf[r, sl] * bbuf[r, sl]
                    return cc

                lax.fori_loop(0, CH, row2, 0)
                for k in range(QW // 16):
                    sl = pl.ds(k * 16, 16)
                    stbuf[0, sl] = stbuf[0, sl] + st[k]
                    stbuf[1, sl] = stbuf[1, sl] + st[4 + k]
                pltpu.sync_copy(dbuf,
                                ep_hbm.at[pl.ds(base, CH), pl.ds(col, QW)])
                pltpu.sync_copy(ebuf, acc_sig.at[didx], add=True)
                pltpu.sync_copy(cbuf, acc_sh.at[didx], add=True)

            return carry

        lax.fori_loop(0, per, chunk_body, 0)
        pltpu.sync_copy(stbuf, acc_st.at[stidx], add=True)
        plsc.subcore_barrier()
        pltpu.sync_copy(acc_sig.at[pl.ds(sid * rows_pt, rows_pt)],
                        sg.at[pl.ds(sid * rows_pt, rows_pt)])
        pltpu.sync_copy(acc_sh.at[pl.ds(sid * rows_pt, rows_pt)],
                        sh.at[pl.ds(sid * rows_pt, rows_pt)])

        @pl.when(sid == 0)
        def _():
            pltpu.sync_copy(acc_st.at[pl.ds(0, 1)], sum_st.at[pl.ds(q, 1)])
            pltpu.sync_copy(acc_st.at[pl.ds(1, 1)], sumsq_st.at[pl.ds(q, 1)])

    for r in range(2):
        @pl.when(cid == 0)
        def _(r=r):
            round_q(2 * r)

        @pl.when(cid == 1)
        def _(r=r):
            round_q(2 * r + 1)


def _sc_fused(src, dst, dh_q, eh_q, bh_q, ce):
    E = src.shape[0]
    N = dh_q[0].shape[0]
    D = NQ * QW
    unit = NS * 8
    NP = ((N + unit - 1) // unit) * unit
    CH = 64
    q_acc = jax.ShapeDtypeStruct((NP, QW), f32)
    sk = pl.kernel(
        functools.partial(_sc_fused_body, NP=NP, E=E, CH=CH),
        out_type=([jax.ShapeDtypeStruct((E, D), f32)]
                  + [q_acc] * NQ + [q_acc] * NQ
                  + [jax.ShapeDtypeStruct((NQ, QW), f32)] * 2),
        mesh=_MESH,
        compiler_params=pltpu.CompilerParams(use_tc_tiling_on_sc=False),
        scratch_types=[
            pltpu.VMEM((CH,), jnp.int32),
            pltpu.VMEM((CH,), jnp.int32),
            pltpu.VMEM((CH, QW), f32),
            pltpu.VMEM((CH, QW), f32),
            pltpu.VMEM((CH, QW), f32),
            pltpu.VMEM((CH, QW), f32),
            pltpu.VMEM((16, QW), f32),
            pltpu.VMEM((16,), jnp.int32),
            pltpu.VMEM((8, QW), f32),
            pltpu.SemaphoreType.DMA,
            pltpu.SemaphoreType.DMA,
            pltpu.SemaphoreType.DMA,
            pltpu.VMEM_SHARED((NP, QW), f32),
            pltpu.VMEM_SHARED((NP, QW), f32),
            pltpu.VMEM_SHARED((16, QW), f32),
        ],
    )
    outs = sk(src, dst, *dh_q, *eh_q, *bh_q, ce)
    ep = outs[0]
    sg_q = outs[1:1 + NQ]
    sh_q = outs[1 + NQ:1 + 2 * NQ]
    sum_st, sumsq_st = outs[1 + 2 * NQ], outs[2 + 2 * NQ]
    return ep, sg_q, sh_q, sum_st, sumsq_st


def _fin_h_a_body(ah_ref, *refs):
    sg_refs = refs[0:NQ]
    sh_refs = refs[NQ:2 * NQ]
    hnew_ref, st_ref = refs[2 * NQ], refs[2 * NQ + 1]
    i = pl.program_id(0)
    ssig = jnp.concatenate([r[...] for r in sg_refs], axis=1)
    ssh = jnp.concatenate([r[...] for r in sh_refs], axis=1)
    hn = ah_ref[...] + ssh / (ssig + 1e-6)
    hnew_ref[...] = hn
    s1 = jnp.sum(hn, axis=0, keepdims=True)
    s2 = jnp.sum(hn * hn, axis=0, keepdims=True)
    blk = jnp.concatenate([s1, s2], axis=0)

    @pl.when(i == 0)
    def _():
        st_ref[...] = blk

    @pl.when(i > 0)
    def _():
        st_ref[...] = st_ref[...] + blk


def _fin_h_a(ah, sg_q, sh_q):
    N, D = ah.shape
    BN = 2000
    row = pl.BlockSpec((BN, D), lambda i: (i, 0))
    qspec = pl.BlockSpec((BN, QW), lambda i: (i, 0))
    return pl.pallas_call(
        _fin_h_a_body,
        grid=(N // BN,),
        in_specs=[row] + [qspec] * (2 * NQ),
        out_specs=[row, pl.BlockSpec((2, D), lambda i: (0, 0))],
        out_shape=[jax.ShapeDtypeStruct((N, D), f32),
                   jax.ShapeDtypeStruct((2, D), f32)],
    )(ah, *sg_q, *sh_q)


def _fin_h_b_body(x_ref, xn_ref, st_ref, g_ref, b_ref, out_ref, *, count):
    mean = st_ref[0:1, :] / count
    var = st_ref[1:2, :] / count - mean * mean
    inv = lax.rsqrt(var + 1e-5)
    xn = (xn_ref[...] - mean) * inv * g_ref[...] + b_ref[...]
    out_ref[...] = x_ref[...] + jnp.maximum(xn, 0.0)


def _fin_h_b(h, h_new, st, gamma, beta):
    N, D = h.shape
    BN = 2000
    row = pl.BlockSpec((BN, D), lambda i: (i, 0))
    return pl.pallas_call(
        functools.partial(_fin_h_b_body, count=float(N)),
        grid=(N // BN,),
        in_specs=[row, row, pl.BlockSpec((2, D), lambda i: (0, 0)),
                  pl.BlockSpec((1, D), lambda i: (0, 0)),
                  pl.BlockSpec((1, D), lambda i: (0, 0))],
        out_specs=row,
        out_shape=jax.ShapeDtypeStruct((N, D), f32),
    )(h, h_new, st, gamma.reshape(1, D), beta.reshape(1, D))


def _fin_e_b_body(e_ref, ep_ref, s1_ref, s2_ref, g_ref, b_ref, out_ref, *,
                  count):
    x = ep_ref[...]
    parts = []
    for q in range(NQ):
        mean = s1_ref[q:q + 1, :] / count
        var = s2_ref[q:q + 1, :] / count - mean * mean
        inv = lax.rsqrt(var + 1e-5)
        parts.append((x[:, q * QW:(q + 1) * QW] - mean) * inv)
    xn = jnp.concatenate(parts, axis=1) * g_ref[...] + b_ref[...]
    out_ref[...] = e_ref[...] + jnp.maximum(xn, 0.0)


def _fin_e_b(e, ep, sum_st, sumsq_st, gamma, beta):
    E, D = e.shape
    BE = 2000
    row = pl.BlockSpec((BE, D), lambda i: (i, 0))
    st_spec = pl.BlockSpec((NQ, QW), lambda i: (0, 0))
    return pl.pallas_call(
        functools.partial(_fin_e_b_body, count=float(E)),
        grid=(E // BE,),
        in_specs=[row, row, st_spec, st_spec,
                  pl.BlockSpec((1, D), lambda i: (0, 0)),
                  pl.BlockSpec((1, D), lambda i: (0, 0))],
        out_specs=row,
        out_shape=jax.ShapeDtypeStruct((E, D), f32),
    )(e, ep, sum_st, sumsq_st, gamma.reshape(1, D), beta.reshape(1, D))


def kernel(h, e, edge_index, WA, bA, WB, bB, WC, bC, WD, bD, WEm, bEm,
           bn_gh, bn_bh, bn_ge, bn_be):
    src = edge_index[0]
    dst = edge_index[1]
    ah, bh_q, dh_q, eh_q = _node_mms(h, WA, bA, WB, bB, WD, bD, WEm, bEm)
    ce = _edge_mm(e, WC, bC)
    ep, sg_q, sh_q, sum_st, sumsq_st = _sc_fused(src, dst, dh_q, eh_q,
                                                 bh_q, ce)
    h_new, st_h = _fin_h_a(ah, sg_q, sh_q)
    h_out = _fin_h_b(h, h_new, st_h, bn_gh, bn_bh)
    e_out = _fin_e_b(e, ep, sum_st, sumsq_st, bn_ge, bn_be)
    return (h_out, e_out)


# speedup vs baseline: 1.6625x; 1.4487x over previous
import functools

import jax
import jax.numpy as jnp
from jax import lax
from jax.experimental import pallas as pl
from jax.experimental.pallas import tpu as pltpu
from jax.experimental.pallas import tpu_sc as plsc

NC, NS, LANES = 2, 16, 16
NW = NC * NS
NQ = 4
QW = 64
CH = 32
G = 4

_MESH = plsc.VectorSubcoreMesh(core_axis_name="c", subcore_axis_name="s")

f32 = jnp.float32


def _node_mm_body(h_ref, wa, ba, wb, bb, wd, bd, wem, bem,
                  ah_ref, bh_ref, dh_ref, eh_ref):
    hb = h_ref[...]
    ah_ref[...] = jnp.dot(hb, wa[...], preferred_element_type=f32) + ba[...]
    bh = jnp.dot(hb, wb[...], preferred_element_type=f32) + bb[...]
    dh = jnp.dot(hb, wd[...], preferred_element_type=f32) + bd[...]
    eh = jnp.dot(hb, wem[...], preferred_element_type=f32) + bem[...]
    for q in range(NQ):
        sl = slice(q * QW, (q + 1) * QW)
        bh_ref[q] = bh[:, sl]
        dh_ref[q] = dh[:, sl]
        eh_ref[q] = eh[:, sl]


def _node_mms(h, WA, bA, WB, bB, WD, bD, WEm, bEm):
    N, D = h.shape
    BN = 2000
    row_spec = pl.BlockSpec((BN, D), lambda i: (i, 0))
    w_spec = pl.BlockSpec((D, D), lambda i: (0, 0))
    b_spec = pl.BlockSpec((1, D), lambda i: (0, 0))
    q_spec = pl.BlockSpec((NQ, BN, QW), lambda i: (0, i, 0))
    q_shape = jax.ShapeDtypeStruct((NQ, N, QW), f32)
    ah, bh_s, dh_s, eh_s = pl.pallas_call(
        _node_mm_body,
        grid=(N // BN,),
        in_specs=[row_spec, w_spec, b_spec, w_spec, b_spec, w_spec, b_spec,
                  w_spec, b_spec],
        out_specs=[row_spec, q_spec, q_spec, q_spec],
        out_shape=[jax.ShapeDtypeStruct((N, D), f32), q_shape, q_shape,
                   q_shape],
    )(h, WA, bA.reshape(1, D), WB, bB.reshape(1, D), WD, bD.reshape(1, D),
      WEm, bEm.reshape(1, D))
    return ah, bh_s, dh_s, eh_s


def _edge_mm_body(e_ref, wc, bc, ce_ref):
    ce_ref[...] = jnp.dot(e_ref[...], wc[...], preferred_element_type=f32) + bc[...]


def _edge_mm(e, WC, bC):
    E, D = e.shape
    BE = 2000
    return pl.pallas_call(
        _edge_mm_body,
        grid=(E // BE,),
        in_specs=[pl.BlockSpec((BE, D), lambda i: (i, 0)),
                  pl.BlockSpec((D, D), lambda i: (0, 0)),
                  pl.BlockSpec((1, D), lambda i: (0, 0))],
        out_specs=pl.BlockSpec((BE, D), lambda i: (i, 0)),
        out_shape=jax.ShapeDtypeStruct((E, D), f32),
    )(e, WC, bC.reshape(1, D))


def _sc_fused_body(src2d, dst2d, dh_s, eh_s, bh_s, ce_hbm,
                   ep_hbm, sg_s, sh_s, sum_st, sumsq_st,
                   ibs0, ibs1, ibd0, ibd1,
                   gd0, gd1, ge0, ge1, gc0, gc1, gb0, gb1,
                   oe0, oe1, og0, og1, op0, op1,
                   dx0, dx1, stbuf, stidx, zbuf,
                   isem0, isem1, gsem0, gsem1, osem0, osem1,
                   acc_sig, acc_sh, acc_st,
                   *, NP, E, PER_T, NPAIRS):
    cid = lax.axis_index("c")
    sid = lax.axis_index("s")
    rows_pt = NP // NS
    nz = rows_pt // 8
    nvalid_ch = E // CH

    IBS = (ibs0, ibs1)
    IBD = (ibd0, ibd1)
    GD = (gd0, gd1)
    GE = (ge0, ge1)
    GC = (gc0, gc1)
    GB = (gb0, gb1)
    OE = (oe0, oe1)
    OG = (og0, og1)
    OP = (op0, op1)
    DX = (dx0, dx1)
    ISEM = (isem0, isem1)
    GSEM = (gsem0, gsem1)
    OSEM = (osem0, osem1)

    tbase = sid * PER_T

    def zrow(r, cc):
        for k in range(QW // 16):
            zbuf[r, pl.ds(k * 16, 16)] = jnp.zeros((16,), f32)
        return cc

    lax.fori_loop(0, 8, zrow, 0)
    stidx[...] = lax.iota(jnp.int32, 16)

    def zero_stbuf():
        def zr(r, cc):
            for k in range(QW // 16):
                stbuf[r, pl.ds(k * 16, 16)] = jnp.zeros((16,), f32)
            return cc
        lax.fori_loop(0, 16, zr, 0)

    def zero_accs():
        def zc(qq, cc):
            pltpu.sync_copy(zbuf, acc_sig.at[pl.ds(sid * rows_pt + qq * 8, 8)])
            pltpu.sync_copy(zbuf, acc_sh.at[pl.ds(sid * rows_pt + qq * 8, 8)])
            return cc
        lax.fori_loop(0, nz, zc, 0)

    def chunk_valid(c):
        return (c < PER_T) & (tbase + c < nvalid_ch)

    def round_q(ri):
        qd = 2 * ri + cid
        col = qd * QW
        dh = dh_s.at[qd]
        eh = eh_s.at[qd]
        bh = bh_s.at[qd]

        zero_accs()
        zero_stbuf()

        @pl.when(sid == 0)
        def _():
            pltpu.sync_copy(stbuf, acc_st)

        plsc.subcore_barrier()

        def ld_idx(g, pb):
            @pl.when(chunk_valid(g * G))
            def _():
                row0 = tbase + g * G
                pltpu.async_copy(src2d.at[pl.ds(row0, G)], IBS[pb], ISEM[pb])
                pltpu.async_copy(dst2d.at[pl.ds(row0, G)], IBD[pb], ISEM[pb])

        def wait_idx(pb):
            pltpu.make_async_copy(src2d.at[pl.ds(0, G)], IBS[pb],
                                  ISEM[pb]).wait()
            pltpu.make_async_copy(dst2d.at[pl.ds(0, G)], IBD[pb],
                                  ISEM[pb]).wait()

        def issue_gathers(c, krow, pb, p):
            @pl.when(chunk_valid(c))
            def _():
                base = (tbase + c) * CH
                srow = IBS[pb].at[krow]
                drow = IBD[pb].at[krow]
                pltpu.async_copy(dh.at[srow], GD[p], GSEM[p])
                pltpu.async_copy(eh.at[drow], GE[p], GSEM[p])
                pltpu.async_copy(ce_hbm.at[pl.ds(base, CH), pl.ds(col, QW)],
                                 GC[p], GSEM[p])
                pltpu.async_copy(bh.at[srow], GB[p], GSEM[p])

        def wait_gathers(p):
            for _ in range(4):
                pltpu.make_async_copy(dh.at[pl.ds(0, CH)], GD[p],
                                      GSEM[p]).wait()

        def wait_outs(p):
            pltpu.make_async_copy(dh.at[pl.ds(0, CH)], OE[p],
                                  OSEM[p]).wait()

        def compute_chunk(c, p):
            def row(r, st):
                nsum = []
                nsq = []
                for k in range(QW // 16):
                    sl = pl.ds(k * 16, 16)
                    epv = GD[p][r, sl] + GE[p][r, sl] + GC[p][r, sl]
                    OE[p][r, sl] = epv
                    nsum.append(st[k] + epv)
                    nsq.append(st[4 + k] + epv * epv)
                    sig = 1.0 / (1.0 + jnp.exp(-epv))
                    OG[p][r, sl] = sig
                    OP[p][r, sl] = sig * GB[p][r, sl]
                return tuple(nsum) + tuple(nsq)

            z16 = jnp.zeros((16,), f32)
            st = lax.fori_loop(0, CH, row, (z16,) * 8)
            for k in range(QW // 16):
                sl = pl.ds(k * 16, 16)
                stbuf[0, sl] = stbuf[0, sl] + st[k]
                stbuf[1, sl] = stbuf[1, sl] + st[4 + k]
            base = (tbase + c) * CH
            pltpu.async_copy(OE[p], ep_hbm.at[pl.ds(base, CH),
                                              pl.ds(col, QW)], OSEM[p])
            pltpu.sync_copy(OG[p], acc_sig.at[DX[p]], add=True)
            pltpu.sync_copy(OP[p], acc_sh.at[DX[p]], add=True)

        def step(gg, half, k):
            g = 2 * gg + half
            pb = half
            p = k % 2
            c = g * G + k

            @pl.when(chunk_valid(c))
            def _():
                wait_gathers(p)

                @pl.when(c >= 2)
                def _():
                    wait_outs(p)

                for t in range(CH // 16):
                    DX[p][pl.ds(t * 16, 16)] = IBD[pb][k, pl.ds(t * 16, 16)]

            if k == G - 1:
                issue_gathers(c + 1, 0, 1 - pb, 1 - p)
            else:
                issue_gathers(c + 1, k + 1, pb, 1 - p)
            if k == G - 1:
                ld_idx(g + 2, pb)

            @pl.when(chunk_valid(c))
            def _():
                compute_chunk(c, p)

        ld_idx(0, 0)
        ld_idx(1, 1)

        @pl.when(chunk_valid(0))
        def _():
            wait_idx(0)
        issue_gathers(0, 0, 0, 0)

        def pair_body(gg, cc):
            for half in range(2):
                for k in range(G):
                    if k == G - 1:
                        @pl.when(chunk_valid((2 * gg + half + 1) * G))
                        def _():
                            wait_idx(1 - half)
                    step(gg, half, k)
            return cc

        lax.fori_loop(0, NPAIRS, pair_body, 0)

        T = jnp.clip(nvalid_ch - tbase, 0, PER_T)

        @pl.when(T >= 2)
        def _():
            wait_outs(0)
            wait_outs(1)

        @pl.when(T == 1)
        def _():
            wait_outs(0)

        pltpu.sync_copy(stbuf, acc_st.at[stidx], add=True)
        plsc.subcore_barrier()
        pltpu.sync_copy(acc_sig.at[pl.ds(sid * rows_pt, rows_pt)],
                        sg_s.at[qd, pl.ds(sid * rows_pt, rows_pt)])
        pltpu.sync_copy(acc_sh.at[pl.ds(sid * rows_pt, rows_pt)],
                        sh_s.at[qd, pl.ds(sid * rows_pt, rows_pt)])

        @pl.when(sid == 0)
        def _():
            pltpu.sync_copy(acc_st.at[pl.ds(0, 1)], sum_st.at[pl.ds(qd, 1)])
            pltpu.sync_copy(acc_st.at[pl.ds(1, 1)], sumsq_st.at[pl.ds(qd, 1)])

    def rbody(ri, cc):
        round_q(ri)
        return cc

    lax.fori_loop(0, 2, rbody, 0)


def _sc_fused(src, dst, dh_s, eh_s, bh_s, ce):
    E = src.shape[0]
    N = dh_s.shape[1]
    D = NQ * QW
    unit = NS * 8
    NP = ((N + unit - 1) // unit) * unit
    nchunks = E // CH
    PER_T = (nchunks + NS - 1) // NS
    ngroups = (PER_T + G - 1) // G
    NPAIRS = (ngroups + 1) // 2
    pad_chunks_pt = (2 * NPAIRS + 2) * G
    E_pad = NS * pad_chunks_pt * CH
    src2d = jnp.pad(src, (0, E_pad - E)).reshape(E_pad // CH, CH)
    dst2d = jnp.pad(dst, (0, E_pad - E)).reshape(E_pad // CH, CH)

    q_acc = jax.ShapeDtypeStruct((NQ, NP, QW), f32)
    idx_t = pltpu.VMEM((G, CH), jnp.int32)
    buf_t = pltpu.VMEM((CH, QW), f32)
    sk = pl.kernel(
        functools.partial(_sc_fused_body, NP=NP, E=E, PER_T=PER_T,
                          NPAIRS=NPAIRS),
        out_type=(jax.ShapeDtypeStruct((E, D), f32), q_acc, q_acc,
                  jax.ShapeDtypeStruct((NQ, QW), f32),
                  jax.ShapeDtypeStruct((NQ, QW), f32)),
        mesh=_MESH,
        compiler_params=pltpu.CompilerParams(use_tc_tiling_on_sc=False),
        scratch_types=[
            idx_t, idx_t, idx_t, idx_t,
            buf_t, buf_t, buf_t, buf_t, buf_t, buf_t, buf_t, buf_t,
            buf_t, buf_t, buf_t, buf_t, buf_t, buf_t,
            pltpu.VMEM((CH,), jnp.int32), pltpu.VMEM((CH,), jnp.int32),
            pltpu.VMEM((16, QW), f32),
            pltpu.VMEM((16,), jnp.int32),
            pltpu.VMEM((8, QW), f32),
            pltpu.SemaphoreType.DMA, pltpu.SemaphoreType.DMA,
            pltpu.SemaphoreType.DMA, pltpu.SemaphoreType.DMA,
            pltpu.SemaphoreType.DMA, pltpu.SemaphoreType.DMA,
            pltpu.VMEM_SHARED((NP, QW), f32),
            pltpu.VMEM_SHARED((NP, QW), f32),
            pltpu.VMEM_SHARED((16, QW), f32),
        ],
    )
    ep, sg_s, sh_s, sum_st, sumsq_st = sk(src2d, dst2d, dh_s, eh_s, bh_s, ce)
    return ep, sg_s, sh_s, sum_st, sumsq_st


def _fin_h_a_body(ah_ref, sg_ref, sh_ref, hnew_ref, st_ref):
    i = pl.program_id(0)
    sg = sg_ref[...]
    sh = sh_ref[...]
    ssig = jnp.concatenate([sg[q] for q in range(NQ)], axis=1)
    ssh = jnp.concatenate([sh[q] for q in range(NQ)], axis=1)
    hn = ah_ref[...] + ssh / (ssig + 1e-6)
    hnew_ref[...] = hn
    s1 = jnp.sum(hn, axis=0, keepdims=True)
    s2 = jnp.sum(hn * hn, axis=0, keepdims=True)
    blk = jnp.concatenate([s1, s2], axis=0)

    @pl.when(i == 0)
    def _():
        st_ref[...] = blk

    @pl.when(i > 0)
    def _():
        st_ref[...] = st_ref[...] + blk


def _fin_h_a(ah, sg_s, sh_s):
    N, D = ah.shape
    BN = 2000
    row = pl.BlockSpec((BN, D), lambda i: (i, 0))
    qspec = pl.BlockSpec((NQ, BN, QW), lambda i: (0, i, 0))
    return pl.pallas_call(
        _fin_h_a_body,
        grid=(N // BN,),
        in_specs=[row, qspec, qspec],
        out_specs=[row, pl.BlockSpec((2, D), lambda i: (0, 0))],
        out_shape=[jax.ShapeDtypeStruct((N, D), f32),
                   jax.ShapeDtypeStruct((2, D), f32)],
    )(ah, sg_s, sh_s)


def _fin_h_b_body(x_ref, xn_ref, st_ref, g_ref, b_ref, out_ref, *, count):
    mean = st_ref[0:1, :] / count
    var = st_ref[1:2, :] / count - mean * mean
    inv = lax.rsqrt(var + 1e-5)
    xn = (xn_ref[...] - mean) * inv * g_ref[...] + b_ref[...]
    out_ref[...] = x_ref[...] + jnp.maximum(xn, 0.0)


def _fin_h_b(h, h_new, st, gamma, beta):
    N, D = h.shape
    BN = 2000
    row = pl.BlockSpec((BN, D), lambda i: (i, 0))
    return pl.pallas_call(
        functools.partial(_fin_h_b_body, count=float(N)),
        grid=(N // BN,),
        in_specs=[row, row, pl.BlockSpec((2, D), lambda i: (0, 0)),
                  pl.BlockSpec((1, D), lambda i: (0, 0)),
                  pl.BlockSpec((1, D), lambda i: (0, 0))],
        out_specs=row,
        out_shape=jax.ShapeDtypeStruct((N, D), f32),
    )(h, h_new, st, gamma.reshape(1, D), beta.reshape(1, D))


def _fin_e_b_body(e_ref, ep_ref, s1_ref, s2_ref, g_ref, b_ref, out_ref, *,
                  count):
    x = ep_ref[...]
    parts = []
    for q in range(NQ):
        mean = s1_ref[q:q + 1, :] / count
        var = s2_ref[q:q + 1, :] / count - mean * mean
        inv = lax.rsqrt(var + 1e-5)
        parts.append((x[:, q * QW:(q + 1) * QW] - mean) * inv)
    xn = jnp.concatenate(parts, axis=1) * g_ref[...] + b_ref[...]
    out_ref[...] = e_ref[...] + jnp.maximum(xn, 0.0)


def _fin_e_b(e, ep, sum_st, sumsq_st, gamma, beta):
    E, D = e.shape
    BE = 2000
    row = pl.BlockSpec((BE, D), lambda i: (i, 0))
    st_spec = pl.BlockSpec((NQ, QW), lambda i: (0, 0))
    return pl.pallas_call(
        functools.partial(_fin_e_b_body, count=float(E)),
        grid=(E // BE,),
        in_specs=[row, row, st_spec, st_spec,
                  pl.BlockSpec((1, D), lambda i: (0, 0)),
                  pl.BlockSpec((1, D), lambda i: (0, 0))],
        out_specs=row,
        out_shape=jax.ShapeDtypeStruct((E, D), f32),
    )(e, ep, sum_st, sumsq_st, gamma.reshape(1, D), beta.reshape(1, D))


def kernel(h, e, edge_index, WA, bA, WB, bB, WC, bC, WD, bD, WEm, bEm,
           bn_gh, bn_bh, bn_ge, bn_be):
    src = edge_index[0]
    dst = edge_index[1]
    ah, bh_s, dh_s, eh_s = _node_mms(h, WA, bA, WB, bB, WD, bD, WEm, bEm)
    ce = _edge_mm(e, WC, bC)
    ep, sg_s, sh_s, sum_st, sumsq_st = _sc_fused(src, dst, dh_s, eh_s,
                                                 bh_s, ce)
    h_new, st_h = _fin_h_a(ah, sg_s, sh_s)
    h_out = _fin_h_b(h, h_new, st_h, bn_gh, bn_bh)
    e_out = _fin_e_b(e, ep, sum_st, sumsq_st, bn_ge, bn_be)
    return (h_out, e_out)
